# baseline (device time: 50437 ns/iter reference)
import jax
import jax.numpy as jnp
from jax import lax
from jax.experimental import pallas as pl
from jax.experimental.pallas import tpu as pltpu

N_DEV = 8
GROUP = 8
CHUNKS = 4


def _cumprod_2d(t):
    rows, n = t.shape
    s = 1
    while s < rows:
        t = t * jnp.concatenate(
            [jnp.ones((s, n), jnp.float32), t[: rows - s, :]], axis=0
        )
        s *= 2
    return t


def _group_scan(xc, gc, n):
    y = xc.reshape(gc, GROUP, n)
    s = 1
    while s < GROUP:
        y = y * jnp.concatenate(
            [jnp.ones((gc, s, n), jnp.float32), y[:, : GROUP - s, :]], axis=1
        )
        s *= 2
    return y


def kernel(x):
    m, n = x.shape
    assert m % (GROUP * CHUNKS) == 0
    g = m // GROUP
    gc = g // CHUNKS
    rc = m // CHUNKS

    def body(x_ref, out_ref, my_tot_ref, totals_ref, send_sems, recv_sems):
        my = lax.axis_index("i")

        tots = []
        for c in range(CHUNKS):
            a = x_ref[c * rc : (c + 1) * rc, :].reshape(gc, GROUP, n)
            p = a[:, :4, :] * a[:, 4:, :]
            p = p[:, :2, :] * p[:, 2:, :]
            p = p[:, 0, :] * p[:, 1, :]
            tots.append(p)
        t2 = _cumprod_2d(jnp.concatenate(tots, axis=0))
        my_tot_ref[:, :] = t2[g - 1 : g, :]

        for j in range(N_DEV):

            @pl.when(my != j)
            def _():
                send = pltpu.make_async_remote_copy(
                    src_ref=my_tot_ref,
                    dst_ref=totals_ref.at[my],
                    send_sem=send_sems.at[j],
                    recv_sem=recv_sems.at[my],
                    device_id=(j,),
                    device_id_type=pl.DeviceIdType.MESH,
                )
                send.start()

        y0 = _group_scan(x_ref[0:rc, :], gc, n)

        for j in range(N_DEV):

            @pl.when(my != j)
            def _():
                send = pltpu.make_async_remote_copy(
                    src_ref=my_tot_ref,
                    dst_ref=totals_ref.at[my],
                    send_sem=send_sems.at[j],
                    recv_sem=recv_sems.at[my],
                    device_id=(j,),
                    device_id_type=pl.DeviceIdType.MESH,
                )
                send.wait_send()
                recv = pltpu.make_async_remote_copy(
                    src_ref=my_tot_ref,
                    dst_ref=totals_ref.at[j],
                    send_sem=send_sems.at[j],
                    recv_sem=recv_sems.at[j],
                    device_id=(j,),
                    device_id_type=pl.DeviceIdType.MESH,
                )
                recv.wait_recv()

        pref = jnp.ones((1, n), jnp.float32)
        for j in range(N_DEV - 1):
            pref = pref * jnp.where(j < my, totals_ref[j, :, :], 1.0)

        ex = pref * jnp.concatenate(
            [jnp.ones((1, n), jnp.float32), t2[: g - 1, :]], axis=0
        )

        out_ref[0:rc, :] = (y0 * ex[0:gc, None, :]).reshape(rc, n)
        for c in range(1, CHUNKS):
            yc = _group_scan(x_ref[c * rc : (c + 1) * rc, :], gc, n)
            out_ref[c * rc : (c + 1) * rc, :] = (
                yc * ex[c * gc : (c + 1) * gc, None, :]
            ).reshape(rc, n)

    return pl.pallas_call(
        body,
        out_shape=jax.ShapeDtypeStruct((m, n), jnp.float32),
        in_specs=[pl.BlockSpec(memory_space=pltpu.VMEM)],
        out_specs=pl.BlockSpec(memory_space=pltpu.VMEM),
        scratch_shapes=[
            pltpu.VMEM((1, n), jnp.float32),
            pltpu.VMEM((N_DEV, 1, n), jnp.float32),
            pltpu.SemaphoreType.DMA((N_DEV,)),
            pltpu.SemaphoreType.DMA((N_DEV,)),
        ],
        compiler_params=pltpu.CompilerParams(
            vmem_limit_bytes=100 * 1024 * 1024,
        ),
    )(x)


# device time: 44878 ns/iter; 1.1239x vs baseline; 1.1239x over previous
import jax
import jax.numpy as jnp
from jax import lax
from jax.experimental import pallas as pl
from jax.experimental.pallas import tpu as pltpu

N_DEV = 8
BLOCK = 512


def _colwise_prod(t):
    rows = t.shape[0]
    while rows > 1:
        half = rows // 2
        t = t[:half, :] * t[half : 2 * half, :]
        rows = half
    return t


def _block_cumprod(c):
    rows, n = c.shape
    s = 1
    while s < rows:
        shifted = jnp.concatenate(
            [jnp.ones((s, n), jnp.float32), c[: rows - s, :]], axis=0
        )
        c = c * shifted
        s *= 2
    return c


def kernel(x):
    m, n = x.shape
    assert m % BLOCK == 0

    def body(x_ref, out_ref, my_tot_ref, totals_ref, send_sems, recv_sems):
        my = lax.axis_index("i")

        my_tot_ref[:, :] = _colwise_prod(x_ref[:, :])

        for j in range(N_DEV):

            @pl.when(my != j)
            def _():
                send = pltpu.make_async_remote_copy(
                    src_ref=my_tot_ref,
                    dst_ref=totals_ref.at[my],
                    send_sem=send_sems.at[j],
                    recv_sem=recv_sems.at[my],
                    device_id=(j,),
                    device_id_type=pl.DeviceIdType.MESH,
                )
                send.start()

        c0 = _block_cumprod(x_ref[0:BLOCK, :])

        for j in range(N_DEV):

            @pl.when(my != j)
            def _():
                send = pltpu.make_async_remote_copy(
                    src_ref=my_tot_ref,
                    dst_ref=totals_ref.at[my],
                    send_sem=send_sems.at[j],
                    recv_sem=recv_sems.at[my],
                    device_id=(j,),
                    device_id_type=pl.DeviceIdType.MESH,
                )
                send.wait_send()
                recv = pltpu.make_async_remote_copy(
                    src_ref=my_tot_ref,
                    dst_ref=totals_ref.at[j],
                    send_sem=send_sems.at[j],
                    recv_sem=recv_sems.at[j],
                    device_id=(j,),
                    device_id_type=pl.DeviceIdType.MESH,
                )
                recv.wait_recv()

        pref = jnp.ones((1, n), jnp.float32)
        for j in range(N_DEV - 1):
            pref = pref * jnp.where(j < my, totals_ref[j, :, :], 1.0)

        c0 = c0 * pref
        out_ref[0:BLOCK, :] = c0

        def blk(b, carry):
            xb = x_ref[pl.ds(b * BLOCK, BLOCK), :]
            cb = _block_cumprod(xb) * carry
            out_ref[pl.ds(b * BLOCK, BLOCK), :] = cb
            return cb[BLOCK - 1 : BLOCK, :]

        lax.fori_loop(1, m // BLOCK, blk, c0[BLOCK - 1 : BLOCK, :])

    return pl.pallas_call(
        body,
        out_shape=jax.ShapeDtypeStruct((m, n), jnp.float32),
        in_specs=[pl.BlockSpec(memory_space=pltpu.VMEM)],
        out_specs=pl.BlockSpec(memory_space=pltpu.VMEM),
        scratch_shapes=[
            pltpu.VMEM((1, n), jnp.float32),
            pltpu.VMEM((N_DEV, 1, n), jnp.float32),
            pltpu.SemaphoreType.DMA((N_DEV,)),
            pltpu.SemaphoreType.DMA((N_DEV,)),
        ],
        compiler_params=pltpu.CompilerParams(
            vmem_limit_bytes=100 * 1024 * 1024,
        ),
    )(x)
